# baseline (speedup 1.0000x reference)
import functools

import jax
from jax import lax
import jax.numpy as jnp
from jax.experimental import pallas as pl
from jax.experimental.pallas import tpu as pltpu
from jax.experimental.pallas import tpu_sc as plsc

NC = 2
NS = 16
NW = NC * NS
L = 16
CW = 128
ROWCAP = 64
NB = 256


def _sc_gather(nodeT, rel_rm, tailT, head_index, rel_type, tail_index):
    D, N = nodeT.shape
    B = head_index.shape[0]
    n_full = N // CW
    tail_w = N - n_full * CW
    tail_tile = n_full % NW
    tail_k = n_full // NW
    quarter = tail_k // 4
    mesh = plsc.VectorSubcoreMesh(core_axis_name="c", subcore_axis_name="s")
    bpw = B // NW

    @functools.partial(
        pl.kernel,
        out_type=[jax.ShapeDtypeStruct((2 * B, D), jnp.float32),
                  jax.ShapeDtypeStruct((B, D), jnp.float32)],
        mesh=mesh,
        compiler_params=pltpu.CompilerParams(use_tc_tiling_on_sc=True,
                                             needs_layout_passes=False),
        scratch_types=[
            pltpu.VMEM((4096,), jnp.int32),
            pltpu.VMEM((2 * B + L,), jnp.int32),
            pltpu.VMEM((2 * B + L,), jnp.int32),
            pltpu.VMEM((NB,), jnp.int32),
            pltpu.VMEM((NB,), jnp.int32),
            pltpu.VMEM((NB + L,), jnp.int32),
            pltpu.VMEM((D, CW), jnp.float32),
            pltpu.VMEM((D, CW), jnp.float32),
            pltpu.VMEM((D, CW), jnp.float32),
            pltpu.VMEM((D, CW), jnp.float32),
            pltpu.VMEM((D, tail_w), jnp.float32),
            pltpu.VMEM((ROWCAP, D), jnp.float32),
            pltpu.VMEM((32, D), jnp.float32),
            pltpu.SemaphoreType.DMA,
            pltpu.SemaphoreType.DMA,
            pltpu.SemaphoreType.DMA,
            pltpu.SemaphoreType.DMA,
            pltpu.SemaphoreType.DMA,
            pltpu.SemaphoreType.DMA,
        ],
    )
    def gather_kernel(nodeT_hbm, rel_hbm, tailT_hbm, hi_hbm, ri_hbm, ti_hbm,
                      ht_out, r_out,
                      stage, plist, sorted_l, hist, curs, start,
                      cb0, cb1, cb2, cb3, tb, rows, relbuf,
                      csem0, csem1, csem2, csem3, wsem, relsem):
        wid = lax.axis_index("s") * NC + lax.axis_index("c")
        iota = lax.iota(jnp.int32, L)
        lane0 = iota == 0

        lcnt = jnp.int32(0)
        for src, pbase in ((hi_hbm, 0), (ti_hbm, B)):
            for piece in range(B // 4096):
                pltpu.sync_copy(src.at[pl.ds(piece * 4096, 4096)], stage)

                def fbody(gg, cnt, pbase=pbase, piece=piece):
                    for u in range(4):
                        g = gg * 4 + u
                        v = stage[pl.ds(g * L, L)]
                        keep = (lax.shift_right_logical(v, 7)
                                & (NW - 1)) == wid
                        posv = iota + (pbase + piece * 4096 + g * L)
                        packed = (lax.shift_left(
                            lax.shift_right_logical(v, 12), 22)
                            | lax.shift_left(v & (CW - 1), 15) | posv)
                        plsc.store_compressed(plist.at[pl.ds(cnt, L)], packed,
                                              mask=keep)
                        cnt = cnt + plsc.all_reduce_population_count(keep)[0]
                    return cnt

                lcnt = lax.fori_loop(0, 4096 // L // 4, fbody, lcnt)
        ngrp = lax.shift_right_logical(lcnt + L - 1, 4)

        zeros = jnp.zeros((L,), jnp.int32)
        for q in range(NB // L):
            hist[pl.ds(q * L, L)] = zeros

        def hbody(g, x):
            pk = plist[pl.ds(g * L, L)]
            bv = lax.shift_right_logical(pk, 22)
            for j in range(L):
                @pl.when((g * L + j) < lcnt)
                def _(j=j):
                    b = jnp.full((L,), bv[j], jnp.int32)
                    cur = plsc.load_gather(hist, [b])
                    plsc.store_scatter(hist, [b], cur + 1, mask=lane0)
            return x
        lax.fori_loop(0, ngrp, hbody, jnp.int32(0))

        run = jnp.int32(0)
        for q in range(NB // L):
            h16 = hist[pl.ds(q * L, L)]
            pc = plsc.cumsum(h16)
            off16 = pc - h16 + run
            start[pl.ds(q * L, L)] = off16
            curs[pl.ds(q * L, L)] = off16
            run = run + pc[L - 1]
        start[pl.ds(NB, L)] = jnp.full((L,), run, jnp.int32)

        def sbody(g, x):
            pk = plist[pl.ds(g * L, L)]
            bv = lax.shift_right_logical(pk, 22)
            for j in range(L):
                @pl.when((g * L + j) < lcnt)
                def _(j=j):
                    b = jnp.full((L,), bv[j], jnp.int32)
                    d = plsc.load_gather(curs, [b])
                    plsc.store_scatter(sorted_l, [d],
                                       jnp.full((L,), pk[j], jnp.int32),
                                       mask=lane0)
                    plsc.store_scatter(curs, [b], d + 1, mask=lane0)
            return x
        lax.fori_loop(0, ngrp, sbody, jnp.int32(0))

        def emit_row(rp_row, buf, rl_j, pos_j):
            slot = rp_row & (ROWCAP - 1)

            @pl.when((rp_row >= ROWCAP) & (slot == 0))
            def _():
                pltpu.make_async_copy(
                    ht_out.at[pl.ds(0, ROWCAP)], rows, wsem).wait()

            cv = jnp.full((L,), rl_j, jnp.int32)
            for q in range(D // L):
                vals = plsc.load_gather(buf, [iota + q * L, cv])
                rows[slot, pl.ds(q * L, L)] = vals
            pltpu.async_copy(rows.at[slot], ht_out.at[pos_j], wsem)

        def process(k, buf, width, enable, rp):
            kb = jnp.full((L,), k, jnp.int32)
            s0 = plsc.load_gather(start, [kb])[0]
            e0 = plsc.load_gather(start, [kb + 1])[0]
            cnt = e0 - s0
            ng = jnp.where(enable,
                           lax.shift_right_logical(cnt + L - 1, 4), 0)

            def gbody(g, rp):
                pk = sorted_l[pl.ds(s0 + g * L, L)]
                rl = jnp.clip(lax.shift_right_logical(pk, 15) & (CW - 1),
                              0, width - 1)
                pv = pk & (2 * B - 1)
                for j in range(L):
                    @pl.when((g * L + j) < cnt)
                    def _(j=j):
                        emit_row(rp + j, buf, rl[j], pv[j])
                return rp + jnp.minimum(cnt - g * L, L)
            return lax.fori_loop(0, ng, gbody, rp)

        def issue(k, buf, sem):
            c = jnp.minimum(wid + NW * k, n_full - 1)
            off = pl.multiple_of(c * CW, CW)
            pltpu.async_copy(nodeT_hbm.at[:, pl.ds(off, CW)], buf, sem)

        bufs = (cb0, cb1, cb2, cb3)
        sems = (csem0, csem1, csem2, csem3)
        for b in range(4):
            issue(b, bufs[b], sems[b])

        def ring(m, rp):
            k0 = 4 * m
            for b in range(4):
                pltpu.make_async_copy(nodeT_hbm.at[:, pl.ds(0, CW)], bufs[b],
                                      sems[b]).wait()
                rp = process(k0 + b, bufs[b], CW, True, rp)

                @pl.when(m < quarter - 1)
                def _(b=b, k0=k0):
                    issue(k0 + b + 4, bufs[b], sems[b])
            return rp

        rp = lax.fori_loop(0, quarter, ring, jnp.int32(0))

        ce = jnp.minimum(wid + NW * tail_k, n_full - 1)
        pltpu.sync_copy(
            nodeT_hbm.at[:, pl.ds(pl.multiple_of(ce * CW, CW), CW)], cb0)
        rp = process(tail_k, cb0, CW, wid < tail_tile, rp)
        pltpu.sync_copy(tailT_hbm, tb)
        rp = process(tail_k, tb, tail_w, wid == tail_tile, rp)

        outstanding = jnp.where(
            rp > 0,
            rp - ROWCAP * lax.shift_right_logical(jnp.maximum(rp - 1, 0), 6),
            0)

        def drain1(_, x):
            pltpu.make_async_copy(ht_out.at[pl.ds(0, 1)],
                                  rows.at[pl.ds(0, 1)], wsem).wait()
            return x
        lax.fori_loop(0, outstanding, drain1, jnp.int32(0))

        base = wid * bpw
        pltpu.sync_copy(ri_hbm.at[pl.ds(base, bpw)], stage.at[pl.ds(0, bpw)])
        for sc_i in range(bpw // 32):
            def rbody(g, x, sc_i=sc_i):
                v = stage[pl.ds(sc_i * 32 + g * L, L)]
                for j in range(L):
                    pltpu.async_copy(rel_hbm.at[v[j]],
                                     relbuf.at[g * L + j], relsem)
                return x
            lax.fori_loop(0, 32 // L, rbody, jnp.int32(0))
            pltpu.make_async_copy(r_out.at[pl.ds(0, 32)], relbuf,
                                  relsem).wait()
            pltpu.sync_copy(relbuf, r_out.at[pl.ds(base + sc_i * 32, 32)])

    return gather_kernel(nodeT, rel_rm, tailT, head_index, rel_type,
                         tail_index)


def _tc_body(h_ref, r_ref, t_ref, o_ref):
    h = h_ref[...]
    r = r_ref[...]
    t = t_ref[...]
    hn = jnp.sqrt(jnp.sum(h * h, axis=1, keepdims=True))
    tn = jnp.sqrt(jnp.sum(t * t, axis=1, keepdims=True))
    hu = h / jnp.maximum(hn, 1e-12)
    tu = t / jnp.maximum(tn, 1e-12)
    d = hu + r - tu
    o_ref[...] = -jnp.sqrt(jnp.sum(d * d, axis=1))


def _tc_score(ht_rows, r_rows):
    B, D = r_rows.shape
    blk = 2048
    nblk = B // blk
    grid = (nblk,)
    h_spec = pl.BlockSpec((blk, D), lambda i: (i, 0))
    t_spec = pl.BlockSpec((blk, D), lambda i: (i + nblk, 0))
    return pl.pallas_call(
        _tc_body,
        grid=grid,
        in_specs=[h_spec, h_spec, t_spec],
        out_specs=pl.BlockSpec((blk,), lambda i: (i,)),
        out_shape=jax.ShapeDtypeStruct((B,), jnp.float32),
    )(ht_rows, r_rows, ht_rows)


def kernel(head_index, rel_type, tail_index, node_emb, rel_emb):
    hi = head_index.astype(jnp.int32)
    ri = rel_type.astype(jnp.int32)
    ti = tail_index.astype(jnp.int32)
    B = hi.shape[0]
    nodeT = node_emb.T
    tail_base = (node_emb.shape[0] // CW) * CW
    tailT = nodeT[:, tail_base:]
    ht_rows, r_rows = _sc_gather(nodeT, rel_emb, tailT, hi, ri, ti)
    return _tc_score(ht_rows, r_rows)


# speedup vs baseline: 1.1346x; 1.1346x over previous
import functools

import jax
from jax import lax
import jax.numpy as jnp
from jax.experimental import pallas as pl
from jax.experimental.pallas import tpu as pltpu
from jax.experimental.pallas import tpu_sc as plsc

NC = 2
NS = 16
NW = NC * NS
L = 16
CW = 128
ROWCAP = 128
NB = 256


def _sc_gather(nodeT, rel_rm, tailT, head_index, rel_type, tail_index):
    D, N = nodeT.shape
    B = head_index.shape[0]
    n_full = N // CW
    tail_w = N - n_full * CW
    tail_tile = n_full % NW
    tail_k = n_full // NW
    half = tail_k // 2
    mesh = plsc.VectorSubcoreMesh(core_axis_name="c", subcore_axis_name="s")
    bpw = B // NW

    @functools.partial(
        pl.kernel,
        out_type=[jax.ShapeDtypeStruct((2 * B, D), jnp.float32),
                  jax.ShapeDtypeStruct((B, D), jnp.float32)],
        mesh=mesh,
        compiler_params=pltpu.CompilerParams(use_tc_tiling_on_sc=True,
                                             needs_layout_passes=False),
        scratch_types=[
            pltpu.VMEM((4096,), jnp.int32),
            pltpu.VMEM((2 * B + L,), jnp.int32),
            pltpu.VMEM((2 * B + L,), jnp.int32),
            pltpu.VMEM((NB,), jnp.int32),
            pltpu.VMEM((NB,), jnp.int32),
            pltpu.VMEM((NB + L,), jnp.int32),
            pltpu.VMEM((D, CW), jnp.float32),
            pltpu.VMEM((D, CW), jnp.float32),
            pltpu.VMEM((D, tail_w), jnp.float32),
            pltpu.VMEM((ROWCAP, D), jnp.float32),
            pltpu.VMEM((64, D), jnp.float32),
            pltpu.SemaphoreType.DMA,
            pltpu.SemaphoreType.DMA,
            pltpu.SemaphoreType.DMA,
            pltpu.SemaphoreType.DMA,
        ],
    )
    def gather_kernel(nodeT_hbm, rel_hbm, tailT_hbm, hi_hbm, ri_hbm, ti_hbm,
                      ht_out, r_out,
                      stage, plist, sorted_l, hist, curs, start,
                      cb0, cb1, tb, rows, relbuf,
                      csem0, csem1, wsem, relsem):
        wid = lax.axis_index("s") * NC + lax.axis_index("c")
        iota = lax.iota(jnp.int32, L)
        lane0 = iota == 0

        lcnt = jnp.int32(0)
        for src, pbase in ((hi_hbm, 0), (ti_hbm, B)):
            for piece in range(B // 4096):
                pltpu.sync_copy(src.at[pl.ds(piece * 4096, 4096)], stage)

                def fbody(gg, cnt, pbase=pbase, piece=piece):
                    for u in range(4):
                        g = gg * 4 + u
                        v = stage[pl.ds(g * L, L)]
                        keep = (lax.shift_right_logical(v, 7)
                                & (NW - 1)) == wid
                        posv = iota + (pbase + piece * 4096 + g * L)
                        packed = (lax.shift_left(
                            lax.shift_right_logical(v, 12), 22)
                            | lax.shift_left(v & (CW - 1), 15) | posv)
                        plsc.store_compressed(plist.at[pl.ds(cnt, L)], packed,
                                              mask=keep)
                        cnt = cnt + plsc.all_reduce_population_count(keep)[0]
                    return cnt

                lcnt = lax.fori_loop(0, 4096 // L // 4, fbody, lcnt)
        ngrp = lax.shift_right_logical(lcnt + L - 1, 4)

        zeros = jnp.zeros((L,), jnp.int32)
        for q in range(NB // L):
            hist[pl.ds(q * L, L)] = zeros

        ones = jnp.full((L,), 1, jnp.int32)

        def hbody(g, x):
            pk = plist[pl.ds(g * L, L)]
            bv = lax.shift_right_logical(pk, 22)
            valid = (iota + g * L) < lcnt
            plsc.addupdate_scatter(hist, [bv], ones, mask=valid)
            return x
        lax.fori_loop(0, ngrp, hbody, jnp.int32(0))

        run = jnp.int32(0)
        for q in range(NB // L):
            h16 = hist[pl.ds(q * L, L)]
            pc = plsc.cumsum(h16)
            off16 = pc - h16 + run
            start[pl.ds(q * L, L)] = off16
            curs[pl.ds(q * L, L)] = off16
            run = run + pc[L - 1]
        start[pl.ds(NB, L)] = jnp.full((L,), run, jnp.int32)

        def sbody(g, x):
            pk = plist[pl.ds(g * L, L)]
            bv = lax.shift_right_logical(pk, 22)
            for j in range(L):
                @pl.when((g * L + j) < lcnt)
                def _(j=j):
                    b = jnp.full((L,), bv[j], jnp.int32)
                    d = plsc.load_gather(curs, [b])
                    plsc.store_scatter(sorted_l, [d],
                                       jnp.full((L,), pk[j], jnp.int32),
                                       mask=lane0)
                    plsc.store_scatter(curs, [b], d + 1, mask=lane0)
            return x
        lax.fori_loop(0, ngrp, sbody, jnp.int32(0))

        def emit_row(rp_row, buf, rl_j, pos_j):
            slot = rp_row & (ROWCAP - 1)

            @pl.when((rp_row >= ROWCAP) & (slot == 0))
            def _():
                pltpu.make_async_copy(
                    ht_out.at[pl.ds(0, ROWCAP)], rows, wsem).wait()

            cv = jnp.full((L,), rl_j, jnp.int32)
            for q in range(D // L):
                vals = plsc.load_gather(buf, [iota + q * L, cv])
                rows[slot, pl.ds(q * L, L)] = vals
            pltpu.async_copy(rows.at[slot], ht_out.at[pos_j], wsem)

        def process(k, buf, width, enable, rp):
            kb = jnp.full((L,), k, jnp.int32)
            s0 = plsc.load_gather(start, [kb])[0]
            e0 = plsc.load_gather(start, [kb + 1])[0]
            cnt = e0 - s0
            ng = jnp.where(enable,
                           lax.shift_right_logical(cnt + L - 1, 4), 0)

            def gbody(g, rp):
                pk = sorted_l[pl.ds(s0 + g * L, L)]
                rl = jnp.clip(lax.shift_right_logical(pk, 15) & (CW - 1),
                              0, width - 1)
                pv = pk & (2 * B - 1)
                for j in range(L):
                    @pl.when((g * L + j) < cnt)
                    def _(j=j):
                        emit_row(rp + j, buf, rl[j], pv[j])
                return rp + jnp.minimum(cnt - g * L, L)
            return lax.fori_loop(0, ng, gbody, rp)

        def issue(k, buf, sem):
            c = jnp.minimum(wid + NW * k, n_full - 1)
            off = pl.multiple_of(c * CW, CW)
            pltpu.async_copy(nodeT_hbm.at[:, pl.ds(off, CW)], buf, sem)

        issue(0, cb0, csem0)
        issue(1, cb1, csem1)

        def ring(m, rp):
            k0 = 2 * m
            pltpu.make_async_copy(nodeT_hbm.at[:, pl.ds(0, CW)], cb0,
                                  csem0).wait()
            rp = process(k0, cb0, CW, True, rp)

            @pl.when(m < half - 1)
            def _():
                issue(k0 + 2, cb0, csem0)

            pltpu.make_async_copy(nodeT_hbm.at[:, pl.ds(0, CW)], cb1,
                                  csem1).wait()
            rp = process(k0 + 1, cb1, CW, True, rp)

            @pl.when(m < half - 1)
            def _():
                issue(k0 + 3, cb1, csem1)
            return rp

        rp = lax.fori_loop(0, half, ring, jnp.int32(0))

        ce = jnp.minimum(wid + NW * tail_k, n_full - 1)
        pltpu.sync_copy(
            nodeT_hbm.at[:, pl.ds(pl.multiple_of(ce * CW, CW), CW)], cb0)
        rp = process(tail_k, cb0, CW, wid < tail_tile, rp)
        pltpu.sync_copy(tailT_hbm, tb)
        rp = process(tail_k, tb, tail_w, wid == tail_tile, rp)

        outstanding = jnp.where(
            rp > 0,
            rp - ROWCAP * lax.shift_right_logical(jnp.maximum(rp - 1, 0), 7),
            0)

        def drain1(_, x):
            pltpu.make_async_copy(ht_out.at[pl.ds(0, 1)],
                                  rows.at[pl.ds(0, 1)], wsem).wait()
            return x
        lax.fori_loop(0, outstanding, drain1, jnp.int32(0))

        base = wid * bpw
        pltpu.sync_copy(ri_hbm.at[pl.ds(base, bpw)], stage.at[pl.ds(0, bpw)])
        for sc_i in range(bpw // 64):
            def rbody(g, x, sc_i=sc_i):
                v = stage[pl.ds(sc_i * 64 + g * L, L)]
                for j in range(L):
                    pltpu.async_copy(rel_hbm.at[v[j]],
                                     relbuf.at[g * L + j], relsem)
                return x
            lax.fori_loop(0, 64 // L, rbody, jnp.int32(0))
            pltpu.make_async_copy(r_out.at[pl.ds(0, 64)], relbuf,
                                  relsem).wait()
            pltpu.sync_copy(relbuf, r_out.at[pl.ds(base + sc_i * 64, 64)])

    return gather_kernel(nodeT, rel_rm, tailT, head_index, rel_type,
                         tail_index)


def _tc_body(h_ref, r_ref, t_ref, o_ref):
    h = h_ref[...]
    r = r_ref[...]
    t = t_ref[...]
    hn = jnp.sqrt(jnp.sum(h * h, axis=1, keepdims=True))
    tn = jnp.sqrt(jnp.sum(t * t, axis=1, keepdims=True))
    hu = h / jnp.maximum(hn, 1e-12)
    tu = t / jnp.maximum(tn, 1e-12)
    d = hu + r - tu
    o_ref[...] = -jnp.sqrt(jnp.sum(d * d, axis=1))


def _tc_score(ht_rows, r_rows):
    B, D = r_rows.shape
    blk = 2048
    nblk = B // blk
    grid = (nblk,)
    h_spec = pl.BlockSpec((blk, D), lambda i: (i, 0))
    t_spec = pl.BlockSpec((blk, D), lambda i: (i + nblk, 0))
    return pl.pallas_call(
        _tc_body,
        grid=grid,
        in_specs=[h_spec, h_spec, t_spec],
        out_specs=pl.BlockSpec((blk,), lambda i: (i,)),
        out_shape=jax.ShapeDtypeStruct((B,), jnp.float32),
    )(ht_rows, r_rows, ht_rows)


def kernel(head_index, rel_type, tail_index, node_emb, rel_emb):
    hi = head_index.astype(jnp.int32)
    ri = rel_type.astype(jnp.int32)
    ti = tail_index.astype(jnp.int32)
    B = hi.shape[0]
    nodeT = node_emb.T
    tail_base = (node_emb.shape[0] // CW) * CW
    tailT = nodeT[:, tail_base:]
    ht_rows, r_rows = _sc_gather(nodeT, rel_emb, tailT, hi, ri, ti)
    return _tc_score(ht_rows, r_rows)


# speedup vs baseline: 1.1546x; 1.0176x over previous
import functools

import jax
from jax import lax
import jax.numpy as jnp
from jax.experimental import pallas as pl
from jax.experimental.pallas import tpu as pltpu
from jax.experimental.pallas import tpu_sc as plsc

NC = 2
NS = 16
NW = NC * NS
L = 16
CW = 128
ROWCAP = 128
NB = 256


def _sc_gather(nodeT, rel_rm, tailT, head_index, rel_type, tail_index):
    D, N = nodeT.shape
    B = head_index.shape[0]
    n_full = N // CW
    tail_w = N - n_full * CW
    tail_tile = n_full % NW
    tail_k = n_full // NW
    half = tail_k // 2
    mesh = plsc.VectorSubcoreMesh(core_axis_name="c", subcore_axis_name="s")
    bpw = B // NW

    @functools.partial(
        pl.kernel,
        out_type=[jax.ShapeDtypeStruct((2 * B, D), jnp.float32),
                  jax.ShapeDtypeStruct((B, D), jnp.float32)],
        mesh=mesh,
        compiler_params=pltpu.CompilerParams(use_tc_tiling_on_sc=True,
                                             needs_layout_passes=False),
        scratch_types=[
            pltpu.VMEM((4096,), jnp.int32),
            pltpu.VMEM((2 * B + L,), jnp.int32),
            pltpu.VMEM((2 * B + L,), jnp.int32),
            pltpu.VMEM((NB,), jnp.int32),
            pltpu.VMEM((NB,), jnp.int32),
            pltpu.VMEM((NB + L,), jnp.int32),
            pltpu.VMEM((D, CW), jnp.float32),
            pltpu.VMEM((D, CW), jnp.float32),
            pltpu.VMEM((D, tail_w), jnp.float32),
            pltpu.VMEM((ROWCAP, D), jnp.float32),
            pltpu.VMEM((64, D), jnp.float32),
            pltpu.SemaphoreType.DMA,
            pltpu.SemaphoreType.DMA,
            pltpu.SemaphoreType.DMA,
            pltpu.SemaphoreType.DMA,
        ],
    )
    def gather_kernel(nodeT_hbm, rel_hbm, tailT_hbm, hi_hbm, ri_hbm, ti_hbm,
                      ht_out, r_out,
                      stage, plist, sorted_l, hist, curs, start,
                      cb0, cb1, tb, rows, relbuf,
                      csem0, csem1, wsem, relsem):
        wid = lax.axis_index("s") * NC + lax.axis_index("c")
        iota = lax.iota(jnp.int32, L)
        lane0 = iota == 0

        lcnt = jnp.int32(0)
        for src, pbase in ((hi_hbm, 0), (ti_hbm, B)):
            for piece in range(B // 4096):
                pltpu.sync_copy(src.at[pl.ds(piece * 4096, 4096)], stage)

                def fbody(gg, cnt, pbase=pbase, piece=piece):
                    for u in range(4):
                        g = gg * 4 + u
                        v = stage[pl.ds(g * L, L)]
                        keep = (lax.shift_right_logical(v, 7)
                                & (NW - 1)) == wid
                        posv = iota + (pbase + piece * 4096 + g * L)
                        packed = (lax.shift_left(
                            lax.shift_right_logical(v, 12), 22)
                            | lax.shift_left(v & (CW - 1), 15) | posv)
                        plsc.store_compressed(plist.at[pl.ds(cnt, L)], packed,
                                              mask=keep)
                        cnt = cnt + plsc.all_reduce_population_count(keep)[0]
                    return cnt

                lcnt = lax.fori_loop(0, 4096 // L // 4, fbody, lcnt)
        ngrp = lax.shift_right_logical(lcnt + L - 1, 4)

        zeros = jnp.zeros((L,), jnp.int32)
        for q in range(NB // L):
            hist[pl.ds(q * L, L)] = zeros

        ones = jnp.full((L,), 1, jnp.int32)

        def hbody(g, x):
            pk = plist[pl.ds(g * L, L)]
            bv = lax.shift_right_logical(pk, 22)
            valid = (iota + g * L) < lcnt
            plsc.addupdate_scatter(hist, [bv], ones, mask=valid)
            return x
        lax.fori_loop(0, ngrp, hbody, jnp.int32(0))

        run = jnp.int32(0)
        for q in range(NB // L):
            h16 = hist[pl.ds(q * L, L)]
            pc = plsc.cumsum(h16)
            off16 = pc - h16 + run
            start[pl.ds(q * L, L)] = off16
            curs[pl.ds(q * L, L)] = off16
            run = run + pc[L - 1]
        start[pl.ds(NB, L)] = jnp.full((L,), run, jnp.int32)

        def sbody(g, x):
            pk = plist[pl.ds(g * L, L)]
            bv = lax.shift_right_logical(pk, 22) & (NB - 1)
            valid = (iota + g * L) < lcnt
            rank = jnp.zeros((L,), jnp.int32)
            for jp in range(L - 1):
                same = (bv == jnp.full((L,), bv[jp], jnp.int32)) & (iota > jp)
                rank = rank + jnp.where(same, 1, 0)
            dests = plsc.load_gather(curs, [bv]) + rank
            plsc.store_scatter(sorted_l, [dests], pk, mask=valid)
            plsc.addupdate_scatter(curs, [bv], ones, mask=valid)
            return x
        lax.fori_loop(0, ngrp, sbody, jnp.int32(0))

        def emit_row(rp_row, buf, rl_j, pos_j):
            slot = rp_row & (ROWCAP - 1)

            @pl.when((rp_row >= ROWCAP) & (slot == 0))
            def _():
                pltpu.make_async_copy(
                    ht_out.at[pl.ds(0, ROWCAP)], rows, wsem).wait()

            cv = jnp.full((L,), rl_j, jnp.int32)
            for q in range(D // L):
                vals = plsc.load_gather(buf, [iota + q * L, cv])
                rows[slot, pl.ds(q * L, L)] = vals
            pltpu.async_copy(rows.at[slot], ht_out.at[pos_j], wsem)

        def process(k, buf, width, enable, rp):
            kb = jnp.full((L,), k, jnp.int32)
            s0 = plsc.load_gather(start, [kb])[0]
            e0 = plsc.load_gather(start, [kb + 1])[0]
            cnt = e0 - s0
            ng = jnp.where(enable,
                           lax.shift_right_logical(cnt + L - 1, 4), 0)

            def gbody(g, rp):
                pk = sorted_l[pl.ds(s0 + g * L, L)]
                rl = jnp.clip(lax.shift_right_logical(pk, 15) & (CW - 1),
                              0, width - 1)
                pv = pk & (2 * B - 1)
                for j in range(L):
                    @pl.when((g * L + j) < cnt)
                    def _(j=j):
                        emit_row(rp + j, buf, rl[j], pv[j])
                return rp + jnp.minimum(cnt - g * L, L)
            return lax.fori_loop(0, ng, gbody, rp)

        def issue(k, buf, sem):
            c = jnp.minimum(wid + NW * k, n_full - 1)
            off = pl.multiple_of(c * CW, CW)
            pltpu.async_copy(nodeT_hbm.at[:, pl.ds(off, CW)], buf, sem)

        issue(0, cb0, csem0)
        issue(1, cb1, csem1)

        def ring(m, rp):
            k0 = 2 * m
            pltpu.make_async_copy(nodeT_hbm.at[:, pl.ds(0, CW)], cb0,
                                  csem0).wait()
            rp = process(k0, cb0, CW, True, rp)

            @pl.when(m < half - 1)
            def _():
                issue(k0 + 2, cb0, csem0)

            pltpu.make_async_copy(nodeT_hbm.at[:, pl.ds(0, CW)], cb1,
                                  csem1).wait()
            rp = process(k0 + 1, cb1, CW, True, rp)

            @pl.when(m < half - 1)
            def _():
                issue(k0 + 3, cb1, csem1)
            return rp

        rp = lax.fori_loop(0, half, ring, jnp.int32(0))

        ce = jnp.minimum(wid + NW * tail_k, n_full - 1)
        pltpu.sync_copy(
            nodeT_hbm.at[:, pl.ds(pl.multiple_of(ce * CW, CW), CW)], cb0)
        rp = process(tail_k, cb0, CW, wid < tail_tile, rp)
        pltpu.sync_copy(tailT_hbm, tb)
        rp = process(tail_k, tb, tail_w, wid == tail_tile, rp)

        outstanding = jnp.where(
            rp > 0,
            rp - ROWCAP * lax.shift_right_logical(jnp.maximum(rp - 1, 0), 7),
            0)

        def drain1(_, x):
            pltpu.make_async_copy(ht_out.at[pl.ds(0, 1)],
                                  rows.at[pl.ds(0, 1)], wsem).wait()
            return x
        lax.fori_loop(0, outstanding, drain1, jnp.int32(0))

        base = wid * bpw
        pltpu.sync_copy(ri_hbm.at[pl.ds(base, bpw)], stage.at[pl.ds(0, bpw)])
        for sc_i in range(bpw // 64):
            def rbody(g, x, sc_i=sc_i):
                v = stage[pl.ds(sc_i * 64 + g * L, L)]
                for j in range(L):
                    pltpu.async_copy(rel_hbm.at[v[j]],
                                     relbuf.at[g * L + j], relsem)
                return x
            lax.fori_loop(0, 64 // L, rbody, jnp.int32(0))
            pltpu.make_async_copy(r_out.at[pl.ds(0, 64)], relbuf,
                                  relsem).wait()
            pltpu.sync_copy(relbuf, r_out.at[pl.ds(base + sc_i * 64, 64)])

    return gather_kernel(nodeT, rel_rm, tailT, head_index, rel_type,
                         tail_index)


def _tc_body(h_ref, r_ref, t_ref, o_ref):
    h = h_ref[...]
    r = r_ref[...]
    t = t_ref[...]
    hn = jnp.sqrt(jnp.sum(h * h, axis=1, keepdims=True))
    tn = jnp.sqrt(jnp.sum(t * t, axis=1, keepdims=True))
    hu = h / jnp.maximum(hn, 1e-12)
    tu = t / jnp.maximum(tn, 1e-12)
    d = hu + r - tu
    o_ref[...] = -jnp.sqrt(jnp.sum(d * d, axis=1))


def _tc_score(ht_rows, r_rows):
    B, D = r_rows.shape
    blk = 2048
    nblk = B // blk
    grid = (nblk,)
    h_spec = pl.BlockSpec((blk, D), lambda i: (i, 0))
    t_spec = pl.BlockSpec((blk, D), lambda i: (i + nblk, 0))
    return pl.pallas_call(
        _tc_body,
        grid=grid,
        in_specs=[h_spec, h_spec, t_spec],
        out_specs=pl.BlockSpec((blk,), lambda i: (i,)),
        out_shape=jax.ShapeDtypeStruct((B,), jnp.float32),
    )(ht_rows, r_rows, ht_rows)


def kernel(head_index, rel_type, tail_index, node_emb, rel_emb):
    hi = head_index.astype(jnp.int32)
    ri = rel_type.astype(jnp.int32)
    ti = tail_index.astype(jnp.int32)
    B = hi.shape[0]
    nodeT = node_emb.T
    tail_base = (node_emb.shape[0] // CW) * CW
    tailT = nodeT[:, tail_base:]
    ht_rows, r_rows = _sc_gather(nodeT, rel_emb, tailT, hi, ri, ti)
    return _tc_score(ht_rows, r_rows)
